# baseline (speedup 1.0000x reference)
import functools

import jax
import jax.numpy as jnp
from jax import lax
from jax.experimental import pallas as pl
from jax.experimental.pallas import tpu as pltpu
from jax.experimental.pallas import tpu_sc as plsc

N = 100000
DIM = 256
DG, DP, DL = 128, 64, 64
R = 128
NB = (N + R - 1) // R
NW = 32
BPW = 25
SLAB = BPW * R
L = 16
NGRP = R // L


def _body(g_h, p_h, l_h, wg_h, wp_h, wl_h, out_h,
          idx_g, idx_p, idx_l, rows0, rows1, wg_v, wp_v, wl_v,
          sem_i, sw0, sw1):
    c = lax.axis_index("c")
    s = lax.axis_index("s")
    w = s * 2 + c
    start = (w * (NB - BPW)) // (NW - 1)
    e0 = start * R

    hs = [
        pltpu.async_copy(g_h.at[pl.ds(e0, SLAB)], idx_g, sem_i),
        pltpu.async_copy(p_h.at[pl.ds(e0, SLAB)], idx_p, sem_i),
        pltpu.async_copy(l_h.at[pl.ds(e0, SLAB)], idx_l, sem_i),
        pltpu.async_copy(wg_h, wg_v, sem_i),
        pltpu.async_copy(wp_h, wp_v, sem_i),
        pltpu.async_copy(wl_h, wl_v, sem_i),
    ]
    for h in hs:
        h.wait()

    iota = lax.iota(jnp.int32, L)
    rowv = [k * L + iota for k in range(NGRP)]

    bufs = (rows0, rows1)
    sws = (sw0, sw1)

    def fill_block(j25):
        buf = bufs[j25 % 2]
        gvs = [idx_g[pl.ds(j25 * R + k * L, L)] for k in range(NGRP)]

        def gcol(j, carry):
            jv = jnp.full((L,), j, jnp.int32)
            for k in range(NGRP):
                v = plsc.load_gather(wg_v, [gvs[k], jv])
                plsc.store_scatter(buf, [rowv[k], jv], v)
            return carry

        lax.fori_loop(0, DG, gcol, 0)

        pvs = [idx_p[pl.ds(j25 * R + k * L, L)] for k in range(NGRP)]

        def pcol(j, carry):
            jv = jnp.full((L,), j, jnp.int32)
            for k in range(NGRP):
                v = plsc.load_gather(wp_v, [pvs[k], jv])
                plsc.store_scatter(buf, [rowv[k], jv + DG], v)
            return carry

        lax.fori_loop(0, DP, pcol, 0)

        lvs = [idx_l[pl.ds(j25 * R + k * L, L)] for k in range(NGRP)]

        def lcol(j, carry):
            jv = jnp.full((L,), j, jnp.int32)
            for k in range(NGRP):
                v = plsc.load_gather(wl_v, [lvs[k], jv])
                plsc.store_scatter(buf, [rowv[k], jv + DG + DP], v)
            return carry

        lax.fori_loop(0, DL, lcol, 0)

    def fire_write(j):
        slot = j % 2
        base = jnp.minimum((start + j) * R, N - R)
        return pltpu.async_copy(bufs[slot], out_h.at[pl.ds(base, R), :],
                                sws[slot])

    wh = [None] * BPW
    for j in range(BPW):
        if j >= 2:
            wh[j - 2].wait()
        fill_block(j)
        wh[j] = fire_write(j)
    wh[BPW - 2].wait()
    wh[BPW - 1].wait()


@jax.jit
def kernel(group, period, ls, Wg, Wp, Wl):
    def layout(x):
        x = x.astype(jnp.int32)
        return jnp.concatenate([x[:(NB - 1) * R], x[N - R:]])

    g1 = layout(group)
    p1 = layout(period)
    l1 = layout(ls)

    mesh = plsc.VectorSubcoreMesh(core_axis_name="c", subcore_axis_name="s")
    run = functools.partial(
        pl.kernel,
        mesh=mesh,
        compiler_params=pltpu.CompilerParams(needs_layout_passes=False),
        out_type=jax.ShapeDtypeStruct((N, DIM), jnp.float32),
        scratch_types=[
            pltpu.VMEM((SLAB,), jnp.int32),
            pltpu.VMEM((SLAB,), jnp.int32),
            pltpu.VMEM((SLAB,), jnp.int32),
            pltpu.VMEM((R, DIM), jnp.float32),
            pltpu.VMEM((R, DIM), jnp.float32),
            pltpu.VMEM((18, DG), jnp.float32),
            pltpu.VMEM((7, DP), jnp.float32),
            pltpu.VMEM((3, DL), jnp.float32),
            pltpu.SemaphoreType.DMA,
            pltpu.SemaphoreType.DMA,
            pltpu.SemaphoreType.DMA,
        ],
    )(_body)
    return run(g1, p1, l1, Wg, Wp, Wl)


# speedup vs baseline: 5.7374x; 5.7374x over previous
import functools

import jax
import jax.numpy as jnp
from jax import lax
from jax.experimental import pallas as pl
from jax.experimental.pallas import tpu as pltpu
from jax.experimental.pallas import tpu_sc as plsc

N = 100000
DIM = 256
DG = 128
R = 128
NB = (N + R - 1) // R
NW = 32
BPW = 25
SLAB = BPW * R
L = 16


def _body(g_h, p_h, l_h, wf_h, out_h,
          idx_g, idx_p, idx_l, idx_pl, rows0, rows1, rows2,
          sem_i, sg0, sg1, sg2, sw0, sw1, sw2):
    c = lax.axis_index("c")
    s = lax.axis_index("s")
    w = s * 2 + c
    start = (w * (NB - BPW)) // (NW - 1)
    e0 = start * R

    h1 = pltpu.async_copy(g_h.at[pl.ds(e0, SLAB)], idx_g, sem_i)
    h2 = pltpu.async_copy(p_h.at[pl.ds(e0, SLAB)], idx_p, sem_i)
    h3 = pltpu.async_copy(l_h.at[pl.ds(e0, SLAB)], idx_l, sem_i)
    h1.wait()
    h2.wait()
    h3.wait()

    for k in range(SLAB // L):
        sl = pl.ds(k * L, L)
        idx_pl[sl] = idx_g[sl] * 21 + idx_p[sl] * 3 + idx_l[sl]

    bufs = (rows0, rows1, rows2)
    sgs = (sg0, sg1, sg2)
    sws = (sw0, sw1, sw2)

    def fire_gathers(j):
        slot = j % 3
        isl = pl.ds(j * R, R)
        a = pltpu.async_copy(wf_h.at[idx_pl.at[isl]], bufs[slot], sgs[slot])
        return (a,)

    def fire_write(j):
        slot = j % 3
        base = jnp.minimum((start + j) * R, N - R)
        return pltpu.async_copy(bufs[slot], out_h.at[pl.ds(base, R), :],
                                sws[slot])

    gh = [None] * BPW
    wh = [None] * BPW
    for j in range(BPW):
        if j >= 3:
            wh[j - 3].wait()
        gh[j] = fire_gathers(j)
        if j >= 2:
            gh[j - 2][0].wait()
            wh[j - 2] = fire_write(j - 2)
    for j in (BPW - 2, BPW - 1):
        gh[j][0].wait()
        wh[j] = fire_write(j)
    wh[BPW - 3].wait()
    wh[BPW - 2].wait()
    wh[BPW - 1].wait()


@jax.jit
def kernel(group, period, ls, Wg, Wp, Wl):
    def layout(x):
        x = x.astype(jnp.int32)
        return jnp.concatenate([x[:(NB - 1) * R], x[N - R:]])

    g1 = layout(group)
    p1 = layout(period)
    l1 = layout(ls)
    wf = jnp.concatenate([
        jnp.repeat(Wg, 21, axis=0),
        jnp.tile(jnp.repeat(Wp, 3, axis=0), (Wg.shape[0], 1)),
        jnp.tile(Wl, (Wg.shape[0] * Wp.shape[0], 1)),
    ], axis=1)

    mesh = plsc.VectorSubcoreMesh(core_axis_name="c", subcore_axis_name="s")
    run = functools.partial(
        pl.kernel,
        mesh=mesh,
        out_type=jax.ShapeDtypeStruct((N, DIM), jnp.float32),
        scratch_types=[
            pltpu.VMEM((SLAB,), jnp.int32),
            pltpu.VMEM((SLAB,), jnp.int32),
            pltpu.VMEM((SLAB,), jnp.int32),
            pltpu.VMEM((SLAB,), jnp.int32),
            pltpu.VMEM((R, DIM), jnp.float32),
            pltpu.VMEM((R, DIM), jnp.float32),
            pltpu.VMEM((R, DIM), jnp.float32),
            pltpu.SemaphoreType.DMA,
            pltpu.SemaphoreType.DMA,
            pltpu.SemaphoreType.DMA,
            pltpu.SemaphoreType.DMA,
            pltpu.SemaphoreType.DMA,
            pltpu.SemaphoreType.DMA,
            pltpu.SemaphoreType.DMA,
        ],
    )(_body)
    return run(g1, p1, l1, wf)
